# Initial kernel scaffold; baseline (speedup 1.0000x reference)
import jax
import jax.numpy as jnp
from jax.experimental import pallas as pl


def kernel(x1, x2, W1, b1, g1, W2, b2, g2, rel_pos_bias, rel_pos_scale, sim_matrix, gates, temperature):
    raise NotImplementedError("write your pallas kernel here")


# speedup vs baseline: 2.9431x; 2.9431x over previous
import math

import jax
import jax.numpy as jnp
from jax.experimental import pallas as pl

B, S, D, P, E = 4, 2048, 1024, 1024, 8
CLAMP_MAX = math.log(1.0 / 0.01)
S_BLK = 512
NS = S // S_BLK
_INV_SQRT2 = 1.0 / math.sqrt(2.0)


def _fc_kernel(x1_ref, x2_ref, w1_ref, b1_ref, g1_ref, w2_ref, b2_ref,
               g2_ref, sum1_ref, sum2_ref):
    s = pl.program_id(1)

    def proc(x_ref, w_ref, b_ref, g_ref):
        h = jnp.dot(x_ref[0], w_ref[...],
                    preferred_element_type=jnp.float32) + b_ref[...]
        ms = jnp.mean(h * h, axis=-1, keepdims=True)
        h = h * jax.lax.rsqrt(ms + 1e-6) * g_ref[...]
        h = 0.5 * h * (1.0 + jax.lax.erf(h * _INV_SQRT2))
        return jnp.sum(h, axis=0, keepdims=True)

    p1 = proc(x1_ref, w1_ref, b1_ref, g1_ref)
    p2 = proc(x2_ref, w2_ref, b2_ref, g2_ref)

    @pl.when(s == 0)
    def _():
        sum1_ref[0] = p1
        sum2_ref[0] = p2

    @pl.when(s != 0)
    def _():
        sum1_ref[0] = sum1_ref[0] + p1
        sum2_ref[0] = sum2_ref[0] + p2


_BR = 8
_EC = 128


def _route_kernel(sum1_ref, sum2_ref, rpb_ref, rps_ref, sim_ref, gates_ref,
                  temp_ref, l_ref, tk_ref):
    rps = rps_ref[0, 0]
    x1m = sum1_ref[...] * (1.0 / S) + rpb_ref[0:1, :] * rps
    x2m = sum2_ref[...] * (1.0 / S) + rpb_ref[1:2, :] * rps
    sim = sim_ref[...]
    raw = (jnp.dot(x1m, sim[0:P, :], preferred_element_type=jnp.float32) +
           jnp.dot(x2m, sim[P:2 * P, :], preferred_element_type=jnp.float32))
    colnorm = jnp.maximum(jnp.sqrt(jnp.sum(sim * sim, axis=0, keepdims=True)),
                          1e-12)
    rowsq = (jnp.sum(x1m * x1m, axis=1, keepdims=True) +
             jnp.sum(x2m * x2m, axis=1, keepdims=True))
    rownorm = jnp.maximum(jnp.sqrt(rowsq), 1e-12)
    scale = jnp.exp(jnp.minimum(temp_ref[0, 0], CLAMP_MAX))
    cos = raw / (rownorm * colnorm)
    logits = jax.nn.sigmoid(cos * scale)
    gate = jax.nn.sigmoid(gates_ref[...] * scale)
    diff = logits - gate
    iota = jax.lax.broadcasted_iota(jnp.int32, (_BR, _EC), 1)
    iota_f = iota.astype(jnp.float32)
    valid = iota < E
    mask_f = jnp.where(jnp.logical_and(diff > 0.0, valid), 1.0, 0.0)
    count = jnp.sum(mask_f, axis=1, keepdims=True)
    count_b = jax.lax.broadcast_in_dim(count, (_BR, _EC), (0, 1))
    diff_m = jnp.where(valid, diff, -1e9)
    maxd = jnp.max(diff_m, axis=1, keepdims=True)
    maxd_b = jax.lax.broadcast_in_dim(maxd, (_BR, _EC), (0, 1))
    idx = jnp.min(jnp.where(diff_m == maxd_b, iota_f, float(_EC)), axis=1,
                  keepdims=True)
    idx_b = jax.lax.broadcast_in_dim(idx, (_BR, _EC), (0, 1))
    onehot_f = jnp.where(iota_f == idx_b, 1.0, 0.0)
    zero_b = count_b < 0.5
    l_ref[...] = jnp.where(zero_b, onehot_f, mask_f)
    tk_ref[...] = jnp.where(zero_b, 1.0, count_b).astype(jnp.int32)


def kernel(x1, x2, W1, b1, g1, W2, b2, g2, rel_pos_bias, rel_pos_scale,
           sim_matrix, gates, temperature):
    sum1, sum2 = pl.pallas_call(
        _fc_kernel,
        grid=(B, NS),
        in_specs=[
            pl.BlockSpec((1, S_BLK, D), lambda b, s: (b, s, 0)),
            pl.BlockSpec((1, S_BLK, D), lambda b, s: (b, s, 0)),
            pl.BlockSpec((D, P), lambda b, s: (0, 0)),
            pl.BlockSpec((1, P), lambda b, s: (0, 0)),
            pl.BlockSpec((1, P), lambda b, s: (0, 0)),
            pl.BlockSpec((D, P), lambda b, s: (0, 0)),
            pl.BlockSpec((1, P), lambda b, s: (0, 0)),
            pl.BlockSpec((1, P), lambda b, s: (0, 0)),
        ],
        out_specs=[
            pl.BlockSpec((1, 1, P), lambda b, s: (b, 0, 0)),
            pl.BlockSpec((1, 1, P), lambda b, s: (b, 0, 0)),
        ],
        out_shape=[
            jax.ShapeDtypeStruct((B, 1, P), jnp.float32),
            jax.ShapeDtypeStruct((B, 1, P), jnp.float32),
        ],
    )(x1, x2, W1, b1.reshape(1, P), g1.reshape(1, P), W2, b2.reshape(1, P),
      g2.reshape(1, P))

    sum1p = jnp.pad(sum1.reshape(B, P), ((0, _BR - B), (0, 0)))
    sum2p = jnp.pad(sum2.reshape(B, P), ((0, _BR - B), (0, 0)))
    sim_p = jnp.pad(sim_matrix, ((0, 0), (0, _EC - E)))
    gates_p = jnp.pad(gates.reshape(1, E), ((0, 0), (0, _EC - E)))

    l, tk = pl.pallas_call(
        _route_kernel,
        out_shape=[
            jax.ShapeDtypeStruct((_BR, _EC), jnp.float32),
            jax.ShapeDtypeStruct((_BR, _EC), jnp.int32),
        ],
    )(sum1p, sum2p, rel_pos_bias, rel_pos_scale.reshape(1, 1), sim_p,
      gates_p, temperature.reshape(1, 1))

    return (l[:B, :E], tk[:B, 0])
